# baseline (speedup 1.0000x reference)
import functools

import numpy as np

import jax
import jax.numpy as jnp
from jax import lax
from jax.experimental import pallas as pl
from jax.experimental.pallas import tpu as pltpu
from jax.experimental.pallas import tpu_sc as plsc

N = 10000
E = 320000
D = 128
N_PAD = 10240
TN = 256
GRID = N_PAD // TN
NC, NS = 2, 16
NW = NC * NS
CHUNK = 64
NB = 3
NCH = 159
NQ = NCH // NB
E_PAD = NW * NCH * CHUNK
ROWS_PER_SUB = N_PAD // NS

_F32 = jnp.float32


def _dot(a, b):
    return jnp.dot(a, b, preferred_element_type=_F32,
                   precision=lax.Precision.HIGHEST)


def _tables0_body(x_ref, we_ref, wk_ref, out_ref):
    xt = x_ref[...]
    q = _dot(xt, wk_ref[0])
    for e in range(3):
        p = _dot(xt, we_ref[e])
        out_ref[:, (2 * e) * D:(2 * e + 1) * D] = p
        out_ref[:, (2 * e + 1) * D:(2 * e + 2) * D] = p + q


def _tables1_body(x_ref, acc_ref, we_ref, wk_ref, out_ref, cur_ref):
    xt = x_ref[...]
    cur = xt + jnp.maximum(acc_ref[0] + acc_ref[1], 0.0)
    cur_ref[...] = cur
    q1 = _dot(cur, wk_ref[1])
    q2 = _dot(xt, wk_ref[2])
    for e in range(3):
        p = _dot(cur, we_ref[e])
        out_ref[:, (3 * e) * D:(3 * e + 1) * D] = p
        out_ref[:, (3 * e + 1) * D:(3 * e + 2) * D] = p + q1
        out_ref[:, (3 * e + 2) * D:(3 * e + 3) * D] = p + q2


def _final_body(cur_ref, acc_ref, out_ref):
    out_ref[...] = cur_ref[...] + jnp.maximum(acc_ref[0] + acc_ref[1], 0.0)


_Z = np.int32(0)
_W_SPEC = pl.BlockSpec((3, D, D), lambda i: (_Z, _Z, _Z))
_X_SPEC = pl.BlockSpec((TN, D), lambda i: (i, _Z))
_ACC_SPEC = pl.BlockSpec((2, TN, D), lambda i: (_Z, i, _Z))

_tables0 = pl.pallas_call(
    _tables0_body,
    grid=(GRID,),
    in_specs=[_X_SPEC, _W_SPEC, _W_SPEC],
    out_specs=pl.BlockSpec((TN, 6 * D), lambda i: (i, _Z)),
    out_shape=jax.ShapeDtypeStruct((N_PAD, 6 * D), _F32),
)

_tables1 = pl.pallas_call(
    _tables1_body,
    grid=(GRID,),
    in_specs=[_X_SPEC, _ACC_SPEC, _W_SPEC, _W_SPEC],
    out_specs=[pl.BlockSpec((TN, 9 * D), lambda i: (i, _Z)), _X_SPEC],
    out_shape=[jax.ShapeDtypeStruct((N_PAD, 9 * D), _F32),
               jax.ShapeDtypeStruct((N_PAD, D), _F32)],
)

_final = pl.pallas_call(
    _final_body,
    grid=(GRID,),
    in_specs=[_X_SPEC, _ACC_SPEC],
    out_specs=_X_SPEC,
    out_shape=jax.ShapeDtypeStruct((N_PAD, D), _F32),
)


@functools.partial(
    pl.kernel,
    mesh=plsc.VectorSubcoreMesh(core_axis_name="c", subcore_axis_name="s"),
    out_type=jax.ShapeDtypeStruct((NC * N_PAD, D), _F32),
    scratch_types=(
        [pltpu.VMEM((NCH, CHUNK), jnp.int32)]
        + [pltpu.VMEM((CHUNK,), jnp.int32)] * NB
        + [pltpu.VMEM((CHUNK, D), _F32)] * NB
        + [pltpu.VMEM_SHARED((N_PAD, D), _F32)]
        + [pltpu.SemaphoreType.DMA] * (3 * NB)
    ),
)
def _sc_edge_accumulate(idx_hbm, dst_hbm, table_hbm, zeros_hbm, out_hbm,
                        idx_v, *rest):
    dstb = rest[:NB]
    buf = rest[NB:2 * NB]
    acc = rest[2 * NB]
    gsem = rest[2 * NB + 1:2 * NB + 1 + NB]
    dsem = rest[2 * NB + 1 + NB:2 * NB + 1 + 2 * NB]
    ssem = rest[2 * NB + 1 + 2 * NB:2 * NB + 1 + 3 * NB]

    def _i32(v):
        return jnp.int32(v)

    c = lax.axis_index("c")
    s = lax.axis_index("s")
    wid = c * NS + s
    r0 = s * ROWS_PER_SUB
    pltpu.sync_copy(zeros_hbm.at[pl.ds(r0, ROWS_PER_SUB)],
                    acc.at[pl.ds(r0, ROWS_PER_SUB)])
    pltpu.sync_copy(idx_hbm.at[wid], idx_v)
    plsc.subcore_barrier()

    def start_fetch(g, b):
        pltpu.async_copy(table_hbm.at[idx_v.at[_i32(g)]], buf[b], gsem[b])
        pltpu.async_copy(dst_hbm.at[wid, _i32(g)], dstb[b], dsem[b])

    def scatter_group(q):
        for b in range(NB):
            pltpu.make_async_copy(table_hbm.at[idx_v.at[_i32(0)]],
                                  buf[b], gsem[b]).wait()
            pltpu.make_async_copy(dst_hbm.at[wid, _i32(0)],
                                  dstb[b], dsem[b]).wait()
            pltpu.async_copy(buf[b], acc.at[dstb[b]], ssem[b], add=True)

    for b in range(NB):
        start_fetch(_i32(b), b)

    def body(q, carry):
        scatter_group(q)
        for b in range(NB):
            pltpu.make_async_copy(buf[b], acc.at[dstb[b]], ssem[b]).wait()
            start_fetch(q * NB + NB + b, b)
        return carry

    lax.fori_loop(jnp.int32(0), jnp.int32(NQ - 1), body, jnp.int32(0))
    scatter_group(_i32(NQ - 1))
    for b in range(NB):
        pltpu.make_async_copy(buf[b], acc.at[dstb[b]], ssem[b]).wait()

    plsc.subcore_barrier()
    pltpu.sync_copy(acc.at[pl.ds(r0, ROWS_PER_SUB)],
                    out_hbm.at[pl.ds(c * N_PAD + r0, ROWS_PER_SUB)])


def kernel(x, Wk, We, edge_index, edge_attr):
    x = x.astype(_F32)
    Wk = Wk.astype(_F32)
    We = We.astype(_F32)
    src = edge_index[0].astype(jnp.int32)
    dst = edge_index[1].astype(jnp.int32)
    ea0 = edge_attr[:, 0].astype(jnp.int32)
    ea1 = edge_attr[:, 1].astype(jnp.int32)

    x_pad = jnp.pad(x, ((0, N_PAD - N), (0, 0)))
    zeros = jnp.zeros((N_PAD, D), _F32)

    perm = jnp.argsort(src)
    src = src[perm]
    dst = dst[perm]
    ea0 = ea0[perm]
    ea1 = ea1[perm]

    idx0 = src * 6 + ea1 * 2 + (ea0 == 1).astype(jnp.int32)
    idx1 = src * 9 + ea1 * 3 + ea0
    pad = E_PAD - E
    idx0_r = jnp.pad(idx0, (0, pad)).reshape(NW, NCH, CHUNK)
    idx1_r = jnp.pad(idx1, (0, pad)).reshape(NW, NCH, CHUNK)
    dst_r = jnp.pad(dst, (0, pad),
                    constant_values=N_PAD - 1).reshape(NW, NCH, CHUNK)

    t0 = _tables0(x_pad, We, Wk).reshape(6 * N_PAD, D)
    acc0 = _sc_edge_accumulate(idx0_r, dst_r, t0, zeros).reshape(2, N_PAD, D)
    t1, cur = _tables1(x_pad, acc0, We, Wk)
    t1 = t1.reshape(9 * N_PAD, D)
    acc1 = _sc_edge_accumulate(idx1_r, dst_r, t1, zeros).reshape(2, N_PAD, D)
    out = _final(cur, acc1)
    return out[:N].astype(jnp.float64)


# speedup vs baseline: 1.2994x; 1.2994x over previous
import functools

import numpy as np

import jax
import jax.numpy as jnp
from jax import lax
from jax.experimental import pallas as pl
from jax.experimental.pallas import tpu as pltpu
from jax.experimental.pallas import tpu_sc as plsc

N = 10000
E = 320000
D = 128
N_PAD = 10240
TN = 256
GRID = N_PAD // TN
NC, NS = 1, 16
NW = NC * NS
CHUNK = 64
NB = 3
NCH = 159
NQ = NCH // NB
EH = E // 2
E_PAD = NW * NCH * CHUNK
ROWS_PER_SUB = N_PAD // NS

_F32 = jnp.float32


def _dot(a, b):
    return jnp.dot(a, b, preferred_element_type=_F32,
                   precision=lax.Precision.HIGHEST)


def _tables0_body(x_ref, we_ref, wk_ref, out_ref):
    xt = x_ref[...]
    q = _dot(xt, wk_ref[0])
    for e in range(3):
        p = _dot(xt, we_ref[e])
        out_ref[2 * e] = p
        out_ref[2 * e + 1] = p + q


def _tables1_body(x_ref, acc_ref, we_ref, wk_ref, out_ref, cur_ref):
    xt = x_ref[...]
    cur = xt + jnp.maximum(acc_ref[0] + acc_ref[1], 0.0)
    cur_ref[...] = cur
    q1 = _dot(cur, wk_ref[1])
    q2 = _dot(xt, wk_ref[2])
    for e in range(3):
        p = _dot(cur, we_ref[e])
        out_ref[3 * e] = p
        out_ref[3 * e + 1] = p + q1
        out_ref[3 * e + 2] = p + q2


def _final_body(cur_ref, acc_ref, out_ref):
    out_ref[...] = cur_ref[...] + jnp.maximum(acc_ref[0] + acc_ref[1], 0.0)


_Z = np.int32(0)
_W_SPEC = pl.BlockSpec((3, D, D), lambda i: (_Z, _Z, _Z))
_X_SPEC = pl.BlockSpec((TN, D), lambda i: (i, _Z))
_ACC_SPEC = pl.BlockSpec((2, TN, D), lambda i: (_Z, i, _Z))

_tables0 = pl.pallas_call(
    _tables0_body,
    grid=(GRID,),
    in_specs=[_X_SPEC, _W_SPEC, _W_SPEC],
    out_specs=pl.BlockSpec((6, TN, D), lambda i: (_Z, i, _Z)),
    out_shape=jax.ShapeDtypeStruct((6, N_PAD, D), _F32),
)

_tables1 = pl.pallas_call(
    _tables1_body,
    grid=(GRID,),
    in_specs=[_X_SPEC, _ACC_SPEC, _W_SPEC, _W_SPEC],
    out_specs=[pl.BlockSpec((9, TN, D), lambda i: (_Z, i, _Z)), _X_SPEC],
    out_shape=[jax.ShapeDtypeStruct((9, N_PAD, D), _F32),
               jax.ShapeDtypeStruct((N_PAD, D), _F32)],
)

_final = pl.pallas_call(
    _final_body,
    grid=(GRID,),
    in_specs=[_X_SPEC, _ACC_SPEC],
    out_specs=_X_SPEC,
    out_shape=jax.ShapeDtypeStruct((N_PAD, D), _F32),
)


@functools.partial(
    pl.kernel,
    mesh=plsc.VectorSubcoreMesh(core_axis_name="c", subcore_axis_name="s",
                                num_cores=NC),
    out_type=jax.ShapeDtypeStruct((NC * N_PAD, D), _F32),
    scratch_types=(
        [pltpu.VMEM((NCH, CHUNK), jnp.int32)]
        + [pltpu.VMEM((CHUNK,), jnp.int32)] * NB
        + [pltpu.VMEM((CHUNK, D), _F32)] * NB
        + [pltpu.VMEM_SHARED((N_PAD, D), _F32)]
        + [pltpu.SemaphoreType.DMA] * (3 * NB)
    ),
)
def _sc_edge_accumulate(idx_hbm, dst_hbm, table_hbm, zeros_hbm, out_hbm,
                        idx_v, *rest):
    dstb = rest[:NB]
    buf = rest[NB:2 * NB]
    acc = rest[2 * NB]
    gsem = rest[2 * NB + 1:2 * NB + 1 + NB]
    dsem = rest[2 * NB + 1 + NB:2 * NB + 1 + 2 * NB]
    ssem = rest[2 * NB + 1 + 2 * NB:2 * NB + 1 + 3 * NB]

    def _i32(v):
        return jnp.int32(v)

    c = lax.axis_index("c")
    s = lax.axis_index("s")
    wid = c * NS + s
    r0 = s * ROWS_PER_SUB
    pltpu.sync_copy(zeros_hbm.at[pl.ds(r0, ROWS_PER_SUB)],
                    acc.at[pl.ds(r0, ROWS_PER_SUB)])
    pltpu.sync_copy(idx_hbm.at[wid], idx_v)
    plsc.subcore_barrier()

    def start_fetch(g, b):
        pltpu.async_copy(table_hbm.at[idx_v.at[_i32(g)]], buf[b], gsem[b])
        pltpu.async_copy(dst_hbm.at[wid, _i32(g)], dstb[b], dsem[b])

    def scatter_group(q):
        for b in range(NB):
            pltpu.make_async_copy(table_hbm.at[idx_v.at[_i32(0)]],
                                  buf[b], gsem[b]).wait()
            pltpu.make_async_copy(dst_hbm.at[wid, _i32(0)],
                                  dstb[b], dsem[b]).wait()
            pltpu.async_copy(buf[b], acc.at[dstb[b]], ssem[b], add=True)

    for b in range(NB):
        start_fetch(_i32(b), b)

    def body(q, carry):
        scatter_group(q)
        for b in range(NB):
            pltpu.make_async_copy(buf[b], acc.at[dstb[b]], ssem[b]).wait()
            start_fetch(q * NB + NB + b, b)
        return carry

    lax.fori_loop(jnp.int32(0), jnp.int32(NQ - 1), body, jnp.int32(0))
    scatter_group(_i32(NQ - 1))
    for b in range(NB):
        pltpu.make_async_copy(buf[b], acc.at[dstb[b]], ssem[b]).wait()

    plsc.subcore_barrier()
    pltpu.sync_copy(acc.at[pl.ds(r0, ROWS_PER_SUB)],
                    out_hbm.at[pl.ds(c * N_PAD + r0, ROWS_PER_SUB)])


def kernel(x, Wk, We, edge_index, edge_attr):
    x = x.astype(_F32)
    Wk = Wk.astype(_F32)
    We = We.astype(_F32)
    src = edge_index[0].astype(jnp.int32)
    dst = edge_index[1].astype(jnp.int32)
    ea0 = edge_attr[:, 0].astype(jnp.int32)
    ea1 = edge_attr[:, 1].astype(jnp.int32)

    x_pad = jnp.pad(x, ((0, N_PAD - N), (0, 0)))
    zeros = jnp.zeros((N_PAD, D), _F32)

    idx0 = (ea1 * 2 + (ea0 == 1).astype(jnp.int32)) * N_PAD + src
    idx1 = (ea1 * 3 + ea0) * N_PAD + src
    pad = E_PAD - EH

    def halves(v, fill):
        a = jnp.pad(v[:EH], (0, pad), constant_values=fill)
        b = jnp.pad(v[EH:], (0, pad), constant_values=fill)
        return a.reshape(NW, NCH, CHUNK), b.reshape(NW, NCH, CHUNK)

    idx0_a, idx0_b = halves(idx0, 0)
    idx1_a, idx1_b = halves(idx1, 0)
    dst_a, dst_b = halves(dst, N_PAD - 1)

    def edge_pass(idx_a, idx_b, table):
        pa = _sc_edge_accumulate(idx_a, dst_a, table, zeros)
        pb = _sc_edge_accumulate(idx_b, dst_b, table, zeros)
        return jnp.stack([pa, pb])

    t0 = _tables0(x_pad, We, Wk).reshape(6 * N_PAD, D)
    acc0 = edge_pass(idx0_a, idx0_b, t0)
    t1, cur = _tables1(x_pad, acc0, We, Wk)
    t1 = t1.reshape(9 * N_PAD, D)
    acc1 = edge_pass(idx1_a, idx1_b, t1)
    out = _final(cur, acc1)
    return out[:N].astype(jnp.float64)


# speedup vs baseline: 1.6904x; 1.3009x over previous
import functools

import numpy as np

import jax
import jax.numpy as jnp
from jax import lax
from jax.experimental import pallas as pl
from jax.experimental.pallas import tpu as pltpu
from jax.experimental.pallas import tpu_sc as plsc

N = 10000
E = 320000
D = 128
N_PAD = 10240
TN = 256
GRID = N_PAD // TN
NC, NS = 2, 16
NW = NC * NS
CHUNK = 64
NB = 3
NCH = 159
NQ = NCH // NB
E_PAD = NW * NCH * CHUNK
ROWS_PER_SUB = N_PAD // NS

_F32 = jnp.float32


def _dot(a, b):
    return jnp.dot(a, b, preferred_element_type=_F32,
                   precision=lax.Precision.HIGHEST)


def _tables0_body(x_ref, we_ref, wk_ref, out_ref):
    xt = x_ref[...]
    q = _dot(xt, wk_ref[0])
    for e in range(3):
        p = _dot(xt, we_ref[e])
        out_ref[2 * e] = p
        out_ref[2 * e + 1] = p + q


def _tables1_body(x_ref, acc_ref, we_ref, wk_ref, out_ref, cur_ref):
    xt = x_ref[...]
    cur = xt + jnp.maximum(acc_ref[0] + acc_ref[1], 0.0)
    cur_ref[...] = cur
    q1 = _dot(cur, wk_ref[1])
    q2 = _dot(xt, wk_ref[2])
    for e in range(3):
        p = _dot(cur, we_ref[e])
        out_ref[3 * e] = p
        out_ref[3 * e + 1] = p + q1
        out_ref[3 * e + 2] = p + q2


def _final_body(cur_ref, acc_ref, out_ref):
    out_ref[...] = cur_ref[...] + jnp.maximum(acc_ref[0] + acc_ref[1], 0.0)


_Z = np.int32(0)
_W_SPEC = pl.BlockSpec((3, D, D), lambda i: (_Z, _Z, _Z))
_X_SPEC = pl.BlockSpec((TN, D), lambda i: (i, _Z))
_ACC_SPEC = pl.BlockSpec((2, TN, D), lambda i: (_Z, i, _Z))

_tables0 = pl.pallas_call(
    _tables0_body,
    grid=(GRID,),
    in_specs=[_X_SPEC, _W_SPEC, _W_SPEC],
    out_specs=pl.BlockSpec((6, TN, D), lambda i: (_Z, i, _Z)),
    out_shape=jax.ShapeDtypeStruct((6, N_PAD, D), _F32),
)

_tables1 = pl.pallas_call(
    _tables1_body,
    grid=(GRID,),
    in_specs=[_X_SPEC, _ACC_SPEC, _W_SPEC, _W_SPEC],
    out_specs=[pl.BlockSpec((9, TN, D), lambda i: (_Z, i, _Z)), _X_SPEC],
    out_shape=[jax.ShapeDtypeStruct((9, N_PAD, D), _F32),
               jax.ShapeDtypeStruct((N_PAD, D), _F32)],
)

_final = pl.pallas_call(
    _final_body,
    grid=(GRID,),
    in_specs=[_X_SPEC, _ACC_SPEC],
    out_specs=_X_SPEC,
    out_shape=jax.ShapeDtypeStruct((N_PAD, D), _F32),
)


@functools.partial(
    pl.kernel,
    mesh=plsc.VectorSubcoreMesh(core_axis_name="c", subcore_axis_name="s"),
    out_type=jax.ShapeDtypeStruct((NC * N_PAD, D), _F32),
    scratch_types=(
        [pltpu.VMEM((NCH, CHUNK), jnp.int32)]
        + [pltpu.VMEM((CHUNK,), jnp.int32)] * NB
        + [pltpu.VMEM((CHUNK, D), _F32)] * NB
        + [pltpu.VMEM_SHARED((N_PAD, D), _F32)]
        + [pltpu.SemaphoreType.DMA] * (3 * NB)
    ),
)
def _sc_edge_accumulate(idx_hbm, dst_hbm, table_hbm, zeros_hbm, out_hbm,
                        idx_v, *rest):
    dstb = rest[:NB]
    buf = rest[NB:2 * NB]
    acc = rest[2 * NB]
    gsem = rest[2 * NB + 1:2 * NB + 1 + NB]
    dsem = rest[2 * NB + 1 + NB:2 * NB + 1 + 2 * NB]
    ssem = rest[2 * NB + 1 + 2 * NB:2 * NB + 1 + 3 * NB]

    def _i32(v):
        return jnp.int32(v)

    c = lax.axis_index("c")
    s = lax.axis_index("s")
    wid = c * NS + s
    r0 = s * ROWS_PER_SUB
    pltpu.sync_copy(zeros_hbm.at[pl.ds(r0, ROWS_PER_SUB)],
                    acc.at[pl.ds(r0, ROWS_PER_SUB)])
    pltpu.sync_copy(idx_hbm.at[wid], idx_v)
    plsc.subcore_barrier()

    def start_fetch(g, b):
        pltpu.async_copy(table_hbm.at[idx_v.at[_i32(g)]], buf[b], gsem[b])
        pltpu.async_copy(dst_hbm.at[wid, _i32(g)], dstb[b], dsem[b])

    def scatter_group(q):
        for b in range(NB):
            pltpu.make_async_copy(table_hbm.at[idx_v.at[_i32(0)]],
                                  buf[b], gsem[b]).wait()
            pltpu.make_async_copy(dst_hbm.at[wid, _i32(0)],
                                  dstb[b], dsem[b]).wait()
            pltpu.async_copy(buf[b], acc.at[dstb[b]], ssem[b], add=True)

    for b in range(NB):
        start_fetch(_i32(b), b)

    def body(q, carry):
        scatter_group(q)
        for b in range(NB):
            pltpu.make_async_copy(buf[b], acc.at[dstb[b]], ssem[b]).wait()
            start_fetch(q * NB + NB + b, b)
        return carry

    lax.fori_loop(jnp.int32(0), jnp.int32(NQ - 1), body, jnp.int32(0))
    scatter_group(_i32(NQ - 1))
    for b in range(NB):
        pltpu.make_async_copy(buf[b], acc.at[dstb[b]], ssem[b]).wait()

    plsc.subcore_barrier()
    pltpu.sync_copy(acc.at[pl.ds(r0, ROWS_PER_SUB)],
                    out_hbm.at[pl.ds(c * N_PAD + r0, ROWS_PER_SUB)])


def kernel(x, Wk, We, edge_index, edge_attr):
    x = x.astype(_F32)
    Wk = Wk.astype(_F32)
    We = We.astype(_F32)
    src = edge_index[0].astype(jnp.int32)
    dst = edge_index[1].astype(jnp.int32)
    ea0 = edge_attr[:, 0].astype(jnp.int32)
    ea1 = edge_attr[:, 1].astype(jnp.int32)

    x_pad = jnp.pad(x, ((0, N_PAD - N), (0, 0)))
    zeros = jnp.zeros((N_PAD, D), _F32)

    idx0 = (ea1 * 2 + (ea0 == 1).astype(jnp.int32)) * N_PAD + src
    idx1 = (ea1 * 3 + ea0) * N_PAD + src
    pad = E_PAD - E
    idx0_r = jnp.pad(idx0, (0, pad)).reshape(NW, NCH, CHUNK)
    idx1_r = jnp.pad(idx1, (0, pad)).reshape(NW, NCH, CHUNK)
    dst_r = jnp.pad(dst, (0, pad),
                    constant_values=N_PAD - 1).reshape(NW, NCH, CHUNK)

    t0 = _tables0(x_pad, We, Wk).reshape(6 * N_PAD, D)
    acc0 = _sc_edge_accumulate(idx0_r, dst_r, t0, zeros).reshape(2, N_PAD, D)
    t1, cur = _tables1(x_pad, acc0, We, Wk)
    t1 = t1.reshape(9 * N_PAD, D)
    acc1 = _sc_edge_accumulate(idx1_r, dst_r, t1, zeros).reshape(2, N_PAD, D)
    out = _final(cur, acc1)
    return out[:N].astype(jnp.float64)
